# baseline (speedup 1.0000x reference)
import functools

import jax
import jax.numpy as jnp
from jax import lax
from jax.experimental import pallas as pl
from jax.experimental.pallas import tpu as pltpu
from jax.experimental.pallas import tpu_sc as plsc

_NC = 2
_NS = 16
_NW = _NC * _NS
_C = 64


@functools.partial(jax.jit, static_argnums=(5, 6, 7, 8))
def _scores(center, pos, negf, in_table, out_table, V, D, B, NNEG):
    BPW = B // _NW
    NCH = BPW // _C
    mesh = plsc.VectorSubcoreMesh(
        core_axis_name="c", subcore_axis_name="s",
        num_cores=_NC, num_subcores=_NS)

    @functools.partial(
        pl.kernel,
        out_type=(
            jax.ShapeDtypeStruct((B,), jnp.float32),
            jax.ShapeDtypeStruct((NNEG, B), jnp.float32),
        ),
        mesh=mesh,
        scratch_types=[
            pltpu.VMEM((_C,), jnp.int32),
            pltpu.VMEM((_C,), jnp.int32),
            pltpu.VMEM((_C * NNEG,), jnp.int32),
            pltpu.VMEM((_C, D), jnp.float32),
            pltpu.VMEM((_C, D), jnp.float32),
            pltpu.VMEM((_C * NNEG, D), jnp.float32),
            pltpu.VMEM((_C,), jnp.float32),
            pltpu.VMEM((NNEG, _C), jnp.float32),
            pltpu.SemaphoreType.DMA,
            pltpu.SemaphoreType.DMA,
            pltpu.SemaphoreType.DMA,
        ],
        compiler_params=pltpu.CompilerParams(use_tc_tiling_on_sc=False,
                                             needs_layout_passes=False),
    )
    def score_k(center_h, pos_h, negf_h, in_t, out_t,
                pos_out_h, neg_out_h,
                c_idx, p_idx, n_idx, c_rows, p_rows, n_rows,
                pos_loc, neg_loc, s1, s2, s3):
        wid = lax.axis_index("s") * _NC + lax.axis_index("c")
        base = wid * BPW

        def chunk_step(ch, carry):
            cb = base + ch * _C
            pltpu.sync_copy(center_h.at[pl.ds(cb, _C)], c_idx)
            pltpu.sync_copy(pos_h.at[pl.ds(cb, _C)], p_idx)
            pltpu.sync_copy(negf_h.at[pl.ds(cb * NNEG, _C * NNEG)], n_idx)
            a1 = pltpu.async_copy(in_t.at[c_idx], c_rows, s1)
            a2 = pltpu.async_copy(out_t.at[p_idx], p_rows, s2)
            a3 = pltpu.async_copy(out_t.at[n_idx], n_rows, s3)
            a1.wait()
            a2.wait()
            a3.wait()
            for g in range(_C // 16):
                rows16 = lax.iota(jnp.int32, 16) + (g * 16)
                nrow0 = rows16 * NNEG

                def dstep(d, accs):
                    cols = jnp.full((16,), 0, jnp.int32) + d
                    c_d = plsc.load_gather(c_rows, [rows16, cols])
                    p_d = plsc.load_gather(p_rows, [rows16, cols])
                    out = [accs[0] + c_d * p_d]
                    for n in range(NNEG):
                        nv = plsc.load_gather(n_rows, [nrow0 + n, cols])
                        out.append(accs[1 + n] + c_d * nv)
                    return tuple(out)

                z = jnp.zeros((16,), jnp.float32)
                accs = lax.fori_loop(0, D, dstep, (z,) * (NNEG + 1),
                                     unroll=2)
                pos_loc[pl.ds(g * 16, 16)] = accs[0]
                for n in range(NNEG):
                    neg_loc[n, pl.ds(g * 16, 16)] = accs[1 + n]
            pltpu.sync_copy(pos_loc, pos_out_h.at[pl.ds(cb, _C)])
            for n in range(NNEG):
                pltpu.sync_copy(neg_loc.at[n], neg_out_h.at[n, pl.ds(cb, _C)])
            return carry

        lax.fori_loop(0, NCH, chunk_step, 0)

    return score_k(center, pos, negf, in_table, out_table)


@functools.partial(jax.jit, static_argnums=(2,))
def _loss(pos_s, neg_s, B):
    def body(p_ref, n_ref, out_ref):
        p = p_ref[...]
        n = n_ref[...]

        def ls(x):
            return jnp.minimum(x, 0.0) - jnp.log(1.0 + jnp.exp(-jnp.abs(x)))

        out_ref[0, 0] = -(jnp.sum(ls(p)) + jnp.sum(ls(-n))) / B

    out = pl.pallas_call(
        body,
        out_specs=pl.BlockSpec(memory_space=pltpu.SMEM),
        out_shape=jax.ShapeDtypeStruct((1, 1), jnp.float32),
    )(pos_s.reshape(128, B // 128), neg_s)
    return out[0, 0]


def kernel(center, pos_context, neg_context, in_table, out_table):
    B = center.shape[0]
    NNEG = neg_context.shape[1]
    V, D = in_table.shape
    c32 = center.astype(jnp.int32)
    p32 = pos_context.astype(jnp.int32)
    n32 = neg_context.astype(jnp.int32).reshape(B * NNEG)
    pos_s, neg_s = _scores(c32, p32, n32, in_table, out_table, V, D, B, NNEG)
    return _loss(pos_s, neg_s, B)


# speedup vs baseline: 1.2834x; 1.2834x over previous
import functools

import jax
import jax.numpy as jnp
from jax import lax
from jax.experimental import pallas as pl
from jax.experimental.pallas import tpu as pltpu
from jax.experimental.pallas import tpu_sc as plsc

_NC = 2
_NS = 16
_NW = _NC * _NS
_C = 64


@functools.partial(jax.jit, static_argnums=(5, 6, 7, 8))
def _scores(center, pos, negf, in_table, out_table, V, D, B, NNEG):
    BPW = B // _NW
    NCH = BPW // _C
    mesh = plsc.VectorSubcoreMesh(
        core_axis_name="c", subcore_axis_name="s",
        num_cores=_NC, num_subcores=_NS)

    @functools.partial(
        pl.kernel,
        out_type=(
            jax.ShapeDtypeStruct((B,), jnp.float32),
            jax.ShapeDtypeStruct((NNEG, B), jnp.float32),
        ),
        mesh=mesh,
        scratch_types=[
            pltpu.VMEM((_C,), jnp.int32),
            pltpu.VMEM((_C,), jnp.int32),
            pltpu.VMEM((_C * NNEG,), jnp.int32),
            pltpu.VMEM((_C, D), jnp.float32),
            pltpu.VMEM((_C, D), jnp.float32),
            pltpu.VMEM((_C * NNEG, D), jnp.float32),
            pltpu.VMEM((_C,), jnp.float32),
            pltpu.VMEM((NNEG, _C), jnp.float32),
            pltpu.SemaphoreType.DMA,
            pltpu.SemaphoreType.DMA,
            pltpu.SemaphoreType.DMA,
        ],
        compiler_params=pltpu.CompilerParams(use_tc_tiling_on_sc=False,
                                             needs_layout_passes=False),
    )
    def score_k(center_h, pos_h, negf_h, in_t, out_t,
                pos_out_h, neg_out_h,
                c_idx, p_idx, n_idx, c_rows, p_rows, n_rows,
                pos_loc, neg_loc, s1, s2, s3):
        wid = lax.axis_index("s") * _NC + lax.axis_index("c")
        base = wid * BPW

        def chunk_step(ch, carry):
            cb = base + ch * _C
            pltpu.sync_copy(center_h.at[pl.ds(cb, _C)], c_idx)
            pltpu.sync_copy(pos_h.at[pl.ds(cb, _C)], p_idx)
            pltpu.sync_copy(negf_h.at[pl.ds(cb * NNEG, _C * NNEG)], n_idx)
            a1 = pltpu.async_copy(in_t.at[c_idx], c_rows, s1)
            a2 = pltpu.async_copy(out_t.at[p_idx], p_rows, s2)
            a3 = pltpu.async_copy(out_t.at[n_idx], n_rows, s3)
            a1.wait()
            a2.wait()
            a3.wait()
            for g in range(_C // 16):
                rows16 = lax.iota(jnp.int32, 16) + (g * 16)
                nrow0 = rows16 * NNEG

                def dstep(d, accs):
                    cols = (lax.iota(jnp.int32, 16) + d) & (D - 1)
                    c_d = plsc.load_gather(c_rows, [rows16, cols])
                    p_d = plsc.load_gather(p_rows, [rows16, cols])
                    out = [accs[0] + c_d * p_d]
                    for n in range(NNEG):
                        nv = plsc.load_gather(n_rows, [nrow0 + n, cols])
                        out.append(accs[1 + n] + c_d * nv)
                    return tuple(out)

                z = jnp.zeros((16,), jnp.float32)
                accs = lax.fori_loop(0, D, dstep, (z,) * (NNEG + 1),
                                     unroll=2)
                pos_loc[pl.ds(g * 16, 16)] = accs[0]
                for n in range(NNEG):
                    neg_loc[n, pl.ds(g * 16, 16)] = accs[1 + n]
            pltpu.sync_copy(pos_loc, pos_out_h.at[pl.ds(cb, _C)])
            for n in range(NNEG):
                pltpu.sync_copy(neg_loc.at[n], neg_out_h.at[n, pl.ds(cb, _C)])
            return carry

        lax.fori_loop(0, NCH, chunk_step, 0)

    return score_k(center, pos, negf, in_table, out_table)


@functools.partial(jax.jit, static_argnums=(2,))
def _loss(pos_s, neg_s, B):
    def body(p_ref, n_ref, out_ref):
        p = p_ref[...]
        n = n_ref[...]

        def ls(x):
            return jnp.minimum(x, 0.0) - jnp.log(1.0 + jnp.exp(-jnp.abs(x)))

        out_ref[0, 0] = -(jnp.sum(ls(p)) + jnp.sum(ls(-n))) / B

    out = pl.pallas_call(
        body,
        out_specs=pl.BlockSpec(memory_space=pltpu.SMEM),
        out_shape=jax.ShapeDtypeStruct((1, 1), jnp.float32),
    )(pos_s.reshape(128, B // 128), neg_s)
    return out[0, 0]


def kernel(center, pos_context, neg_context, in_table, out_table):
    B = center.shape[0]
    NNEG = neg_context.shape[1]
    V, D = in_table.shape
    c32 = center.astype(jnp.int32)
    p32 = pos_context.astype(jnp.int32)
    n32 = neg_context.astype(jnp.int32).reshape(B * NNEG)
    pos_s, neg_s = _scores(c32, p32, n32, in_table, out_table, V, D, B, NNEG)
    return _loss(pos_s, neg_s, B)


# speedup vs baseline: 1.2891x; 1.0044x over previous
import functools

import jax
import jax.numpy as jnp
from jax import lax
from jax.experimental import pallas as pl
from jax.experimental.pallas import tpu as pltpu
from jax.experimental.pallas import tpu_sc as plsc

_NC = 2
_NS = 16
_NW = _NC * _NS
_C = 32


@functools.partial(jax.jit, static_argnums=(5, 6, 7, 8))
def _scores(center, pos, negf, in_table, out_table, V, D, B, NNEG):
    BPW = B // _NW
    NCH = BPW // _C
    mesh = plsc.VectorSubcoreMesh(
        core_axis_name="c", subcore_axis_name="s",
        num_cores=_NC, num_subcores=_NS)

    @functools.partial(
        pl.kernel,
        out_type=(
            jax.ShapeDtypeStruct((B,), jnp.float32),
            jax.ShapeDtypeStruct((NNEG, B), jnp.float32),
        ),
        mesh=mesh,
        scratch_types=[
            pltpu.VMEM((_C,), jnp.int32),
            pltpu.VMEM((_C,), jnp.int32),
            pltpu.VMEM((_C,), jnp.int32),
            pltpu.VMEM((_C,), jnp.int32),
            pltpu.VMEM((_C * NNEG,), jnp.int32),
            pltpu.VMEM((_C * NNEG,), jnp.int32),
            pltpu.VMEM((_C, D), jnp.float32),
            pltpu.VMEM((_C, D), jnp.float32),
            pltpu.VMEM((_C, D), jnp.float32),
            pltpu.VMEM((_C, D), jnp.float32),
            pltpu.VMEM((_C * NNEG, D), jnp.float32),
            pltpu.VMEM((_C * NNEG, D), jnp.float32),
            pltpu.VMEM((_C,), jnp.float32),
            pltpu.VMEM((NNEG, _C), jnp.float32),
            pltpu.SemaphoreType.DMA,
            pltpu.SemaphoreType.DMA,
            pltpu.SemaphoreType.DMA,
            pltpu.SemaphoreType.DMA,
            pltpu.SemaphoreType.DMA,
            pltpu.SemaphoreType.DMA,
        ],
        compiler_params=pltpu.CompilerParams(use_tc_tiling_on_sc=False,
                                             needs_layout_passes=False),
    )
    def score_k(center_h, pos_h, negf_h, in_t, out_t,
                pos_out_h, neg_out_h,
                ci0, ci1, pi0, pi1, ni0, ni1,
                cr0, cr1, pr0, pr1, nr0, nr1,
                pos_loc, neg_loc,
                s10, s11, s20, s21, s30, s31):
        cis, pis, nis = (ci0, ci1), (pi0, pi1), (ni0, ni1)
        crs, prs, nrs = (cr0, cr1), (pr0, pr1), (nr0, nr1)
        s1s, s2s, s3s = (s10, s11), (s20, s21), (s30, s31)
        wid = lax.axis_index("s") * _NC + lax.axis_index("c")
        base = wid * BPW

        def issue(ch, b):
            cb = base + ch * _C
            pltpu.sync_copy(center_h.at[pl.ds(cb, _C)], cis[b])
            pltpu.sync_copy(pos_h.at[pl.ds(cb, _C)], pis[b])
            pltpu.sync_copy(negf_h.at[pl.ds(cb * NNEG, _C * NNEG)], nis[b])
            pltpu.async_copy(in_t.at[cis[b]], crs[b], s1s[b])
            pltpu.async_copy(out_t.at[pis[b]], prs[b], s2s[b])
            pltpu.async_copy(out_t.at[nis[b]], nrs[b], s3s[b])

        def wait(b):
            pltpu.make_async_copy(in_t.at[cis[b]], crs[b], s1s[b]).wait()
            pltpu.make_async_copy(out_t.at[pis[b]], prs[b], s2s[b]).wait()
            pltpu.make_async_copy(out_t.at[nis[b]], nrs[b], s3s[b]).wait()

        issue(0, 0)
        issue(1, 1)

        def pair_step(i, carry):
            for b in range(2):
                ch = 2 * i + b
                cb = base + ch * _C
                wait(b)
                for g in range(_C // 16):
                    rows16 = lax.iota(jnp.int32, 16) + (g * 16)
                    nrow0 = rows16 * NNEG

                    def dstep(d, accs):
                        cols = (lax.iota(jnp.int32, 16) + d) & (D - 1)
                        c_d = plsc.load_gather(crs[b], [rows16, cols])
                        p_d = plsc.load_gather(prs[b], [rows16, cols])
                        out = [accs[0] + c_d * p_d]
                        for n in range(NNEG):
                            nv = plsc.load_gather(nrs[b], [nrow0 + n, cols])
                            out.append(accs[1 + n] + c_d * nv)
                        return tuple(out)

                    z = jnp.zeros((16,), jnp.float32)
                    accs = lax.fori_loop(0, D, dstep, (z,) * (NNEG + 1),
                                         unroll=2)
                    pos_loc[pl.ds(g * 16, 16)] = accs[0]
                    for n in range(NNEG):
                        neg_loc[n, pl.ds(g * 16, 16)] = accs[1 + n]

                nxt = ch + 2

                @pl.when(nxt < NCH)
                def _():
                    issue(nxt, b)

                pltpu.sync_copy(pos_loc, pos_out_h.at[pl.ds(cb, _C)])
                for n in range(NNEG):
                    pltpu.sync_copy(neg_loc.at[n],
                                    neg_out_h.at[n, pl.ds(cb, _C)])
            return carry

        lax.fori_loop(0, NCH // 2, pair_step, 0)

    return score_k(center, pos, negf, in_table, out_table)


@functools.partial(jax.jit, static_argnums=(2,))
def _loss(pos_s, neg_s, B):
    def body(p_ref, n_ref, out_ref):
        p = p_ref[...]
        n = n_ref[...]

        def ls(x):
            return jnp.minimum(x, 0.0) - jnp.log(1.0 + jnp.exp(-jnp.abs(x)))

        out_ref[0, 0] = -(jnp.sum(ls(p)) + jnp.sum(ls(-n))) / B

    out = pl.pallas_call(
        body,
        out_specs=pl.BlockSpec(memory_space=pltpu.SMEM),
        out_shape=jax.ShapeDtypeStruct((1, 1), jnp.float32),
    )(pos_s.reshape(128, B // 128), neg_s)
    return out[0, 0]


def kernel(center, pos_context, neg_context, in_table, out_table):
    B = center.shape[0]
    NNEG = neg_context.shape[1]
    V, D = in_table.shape
    c32 = center.astype(jnp.int32)
    p32 = pos_context.astype(jnp.int32)
    n32 = neg_context.astype(jnp.int32).reshape(B * NNEG)
    pos_s, neg_s = _scores(c32, p32, n32, in_table, out_table, V, D, B, NNEG)
    return _loss(pos_s, neg_s, B)
